# baseline (speedup 1.0000x reference)
import functools

import jax
import jax.numpy as jnp
from jax import lax
from jax.experimental import pallas as pl
from jax.experimental.pallas import tpu as pltpu
from jax.experimental.pallas import tpu_sc as plsc

_NC, _NS, _L = 2, 16, 16
_NW = _NC * _NS
_B = 16384
_D = 128
_BPW = _B // _NW
_CH = 4
_CHB = _BPW // _CH
_G = _CHB // _L


def _sc_router(idx2d, table, wb):
    mesh = plsc.VectorSubcoreMesh(core_axis_name="c", subcore_axis_name="s")

    @functools.partial(
        pl.kernel,
        out_type=jax.ShapeDtypeStruct((2, _B), jnp.float32),
        mesh=mesh,
        scratch_types=[
            pltpu.VMEM((_CH, _CHB), jnp.int32),
            pltpu.VMEM((_BPW, _D), jnp.float32),
            pltpu.VMEM((528,), jnp.float32),
            pltpu.VMEM((_BPW,), jnp.float32),
            pltpu.VMEM((_BPW,), jnp.float32),
            pltpu.SemaphoreType.DMA,
            pltpu.SemaphoreType.DMA,
            pltpu.SemaphoreType.DMA,
        ],
        compiler_params=pltpu.CompilerParams(needs_layout_passes=False),
    )
    def body(idx_hbm, table_hbm, wb_hbm, out_hbm,
             idx_v, rows_v, wb_v, o0_v, o1_v, sem_a, sem_b, sem_o):
        wid = lax.axis_index("s") * _NC + lax.axis_index("c")
        base = wid * _BPW
        pltpu.sync_copy(idx_hbm.at[pl.ds(wid * _CH, _CH)], idx_v)
        pltpu.sync_copy(wb_hbm, wb_v)

        iota16 = lax.iota(jnp.int32, _L)
        b0 = plsc.load_gather(wb_v, [jnp.full((_L,), 4 * _D, jnp.int32)])
        b1 = plsc.load_gather(wb_v, [jnp.full((_L,), 4 * _D + 1, jnp.int32)])

        sems = [sem_a, sem_b]

        def fire(c):
            return pltpu.async_copy(
                table_hbm.at[idx_v.at[c]],
                rows_v.at[pl.ds(c * _CHB, _CHB)], sems[c % 2])

        cps = {0: fire(0), 1: fire(1)}
        for c in range(_CH):
            cps[c].wait()
            if c + 2 < _CH:
                cps[c + 2] = fire(c + 2)

            rids = [c * _CHB + g * _L + iota16 for g in range(_G)]

            def dbody(s, accs, rids=rids):
                dmod = (iota16 + s) & (_D - 1)
                w0r = wb_v[pl.ds(s, _L)]
                w1r = wb_v[pl.ds(s + 2 * _D, _L)]
                nxt = []
                for g in range(_G):
                    col = plsc.load_gather(rows_v, [rids[g], dmod])
                    nxt.append(accs[2 * g] + col * w0r)
                    nxt.append(accs[2 * g + 1] + col * w1r)
                return tuple(nxt)

            accs = lax.fori_loop(
                0, _D, dbody,
                tuple(jnp.zeros((_L,), jnp.float32) for _ in range(2 * _G)),
                unroll=2)
            for g in range(_G):
                off = c * _CHB + g * _L
                o0_v[pl.ds(off, _L)] = accs[2 * g] + b0
                o1_v[pl.ds(off, _L)] = accs[2 * g + 1] + b1

        cp0 = pltpu.async_copy(o0_v, out_hbm.at[0, pl.ds(base, _BPW)], sem_o)
        cp1 = pltpu.async_copy(o1_v, out_hbm.at[1, pl.ds(base, _BPW)], sem_o)
        cp0.wait()
        cp1.wait()

    return body(idx2d, table, wb)


def kernel(input_ids, table, W, b):
    idx2d = input_ids[:, 0].astype(jnp.int32).reshape(_NW * _CH, _CHB)
    Wf = W.astype(jnp.float32)
    w0, w1 = Wf[:, 0], Wf[:, 1]
    wb = jnp.concatenate([
        w0, w0, w1, w1,
        b.astype(jnp.float32),
        jnp.zeros((14,), jnp.float32),
    ])
    out2 = _sc_router(idx2d, table.astype(jnp.float32), wb)
    return out2.T


# speedup vs baseline: 1.0317x; 1.0317x over previous
import functools

import jax
import jax.numpy as jnp
from jax import lax
from jax.experimental import pallas as pl
from jax.experimental.pallas import tpu as pltpu
from jax.experimental.pallas import tpu_sc as plsc

_NC, _NS, _L = 2, 16, 16
_NW = _NC * _NS
_B = 16384
_D = 128
_BPW = _B // _NW
_CH = 4
_CHB = _BPW // _CH
_G = _CHB // _L


def _sc_router(idx2d, table, wb):
    mesh = plsc.VectorSubcoreMesh(core_axis_name="c", subcore_axis_name="s")

    @functools.partial(
        pl.kernel,
        out_type=jax.ShapeDtypeStruct((2, _B), jnp.float32),
        mesh=mesh,
        scratch_types=[
            pltpu.VMEM((_CH, _CHB), jnp.int32),
            pltpu.VMEM((_BPW, _D), jnp.float32),
            pltpu.VMEM((528,), jnp.float32),
            pltpu.VMEM((_BPW,), jnp.float32),
            pltpu.VMEM((_BPW,), jnp.float32),
            pltpu.SemaphoreType.DMA,
            pltpu.SemaphoreType.DMA,
            pltpu.SemaphoreType.DMA,
        ],
        compiler_params=pltpu.CompilerParams(needs_layout_passes=False),
    )
    def body(idx_hbm, table_hbm, wb_hbm, out_hbm,
             idx_v, rows_v, wb_v, o0_v, o1_v, sem_a, sem_b, sem_o):
        wid = lax.axis_index("s") * _NC + lax.axis_index("c")
        base = wid * _BPW
        pltpu.sync_copy(idx_hbm.at[pl.ds(wid * _CH, _CH)], idx_v)

        sems = [sem_a, sem_b]

        def fire(c):
            return pltpu.async_copy(
                table_hbm.at[idx_v.at[c]],
                rows_v.at[pl.ds(c * _CHB, _CHB)], sems[c % 2])

        cps = {0: fire(0), 1: fire(1)}
        wb_cp = pltpu.async_copy(wb_hbm, wb_v, sem_o)
        wb_cp.wait()

        iota16 = lax.iota(jnp.int32, _L)
        b0 = plsc.load_gather(wb_v, [jnp.full((_L,), 4 * _D, jnp.int32)])
        b1 = plsc.load_gather(wb_v, [jnp.full((_L,), 4 * _D + 1, jnp.int32)])
        for c in range(_CH):
            cps[c].wait()
            if c + 2 < _CH:
                cps[c + 2] = fire(c + 2)

            rids = [c * _CHB + g * _L + iota16 for g in range(_G)]

            def dbody(s, accs, rids=rids):
                dmod = (iota16 + s) & (_D - 1)
                w0r = wb_v[pl.ds(s, _L)]
                w1r = wb_v[pl.ds(s + 2 * _D, _L)]
                nxt = []
                for g in range(_G):
                    col = plsc.load_gather(rows_v, [rids[g], dmod])
                    nxt.append(accs[2 * g] + col * w0r)
                    nxt.append(accs[2 * g + 1] + col * w1r)
                return tuple(nxt)

            accs = lax.fori_loop(
                0, _D, dbody,
                tuple(jnp.zeros((_L,), jnp.float32) for _ in range(2 * _G)),
                unroll=2)
            for g in range(_G):
                off = c * _CHB + g * _L
                o0_v[pl.ds(off, _L)] = accs[2 * g] + b0
                o1_v[pl.ds(off, _L)] = accs[2 * g + 1] + b1

        cp0 = pltpu.async_copy(o0_v, out_hbm.at[0, pl.ds(base, _BPW)], sem_o)
        cp1 = pltpu.async_copy(o1_v, out_hbm.at[1, pl.ds(base, _BPW)], sem_o)
        cp0.wait()
        cp1.wait()

    return body(idx2d, table, wb)


def kernel(input_ids, table, W, b):
    idx2d = input_ids[:, 0].astype(jnp.int32).reshape(_NW * _CH, _CHB)
    Wf = W.astype(jnp.float32)
    w0, w1 = Wf[:, 0], Wf[:, 1]
    wb = jnp.concatenate([
        w0, w0, w1, w1,
        b.astype(jnp.float32),
        jnp.zeros((14,), jnp.float32),
    ])
    out2 = _sc_router(idx2d, table.astype(jnp.float32), wb)
    return out2.T
